# Initial kernel scaffold; baseline (speedup 1.0000x reference)
import jax
import jax.numpy as jnp
from jax.experimental import pallas as pl


def kernel(x, edge_index):
    raise NotImplementedError("write your pallas kernel here")


# speedup vs baseline: 3.3607x; 3.3607x over previous
import functools

import jax
import jax.numpy as jnp
from jax import lax
from jax.experimental import pallas as pl
from jax.experimental.pallas import tpu as pltpu
from jax.experimental.pallas import tpu_sc as plsc

N_NODES = 10000
N_EDGES = 320000
D_FEAT = 128

NW = 32
E_PER_W = N_EDGES // NW
C = 80
NCHUNK = E_PER_W // C
LANES = 16

_mesh = plsc.VectorSubcoreMesh(core_axis_name="c", subcore_axis_name="s")


@functools.partial(
    pl.kernel,
    out_type=jax.ShapeDtypeStruct((N_EDGES, D_FEAT), jnp.float32),
    mesh=_mesh,
    scratch_types=[
        pltpu.VMEM((C,), jnp.int32),
        pltpu.VMEM((C,), jnp.int32),
        pltpu.VMEM((C, D_FEAT), jnp.float32),
        pltpu.VMEM((C, D_FEAT), jnp.float32),
        pltpu.SemaphoreType.DMA,
    ],
)
def _edge_grad(x_hbm, src_hbm, dst_hbm, out_hbm, idx_s, idx_d, rows_s, rows_d, sem):
    wid = lax.axis_index("s") * 2 + lax.axis_index("c")
    base0 = wid * E_PER_W

    def chunk_body(k, carry):
        base = base0 + k * C
        pltpu.sync_copy(src_hbm.at[pl.ds(base, C)], idx_s)
        pltpu.sync_copy(dst_hbm.at[pl.ds(base, C)], idx_d)
        cp_s = pltpu.async_copy(x_hbm.at[idx_s], rows_s, sem)
        cp_d = pltpu.async_copy(x_hbm.at[idx_d], rows_d, sem)
        cp_s.wait()
        cp_d.wait()

        def row_body(r, rcarry):
            for j in range(D_FEAT // LANES):
                sl = pl.ds(j * LANES, LANES)
                rows_d[r, sl] = rows_d[r, sl] - rows_s[r, sl]
            return rcarry

        lax.fori_loop(0, C, row_body, 0, unroll=False)
        pltpu.sync_copy(rows_d, out_hbm.at[pl.ds(base, C)])
        return carry

    lax.fori_loop(0, NCHUNK, chunk_body, 0, unroll=False)


def kernel(x, edge_index):
    src = edge_index[0]
    dst = edge_index[1]
    return _edge_grad(x, src, dst)


# speedup vs baseline: 7.2110x; 2.1457x over previous
import functools

import jax
import jax.numpy as jnp
from jax import lax
from jax.experimental import pallas as pl
from jax.experimental.pallas import tpu as pltpu
from jax.experimental.pallas import tpu_sc as plsc

N_NODES = 10000
N_EDGES = 320000
D_FEAT = 128

NW = 32
E_PER_W = N_EDGES // NW
C = 80
NBUF = 5
NGROUP = E_PER_W // (C * NBUF)

_mesh = plsc.VectorSubcoreMesh(core_axis_name="c", subcore_axis_name="s")


def _neg_body(x_ref, o_ref):
    o_ref[...] = -x_ref[...]


def _negate(x):
    return pl.pallas_call(
        _neg_body,
        out_shape=jax.ShapeDtypeStruct((N_NODES, D_FEAT), jnp.float32),
    )(x)


_scratch = []
for _b in range(NBUF):
    _scratch += [
        pltpu.VMEM((C,), jnp.int32),
        pltpu.VMEM((C,), jnp.int32),
        pltpu.VMEM((C, D_FEAT), jnp.float32),
        pltpu.SemaphoreType.DMA,
        pltpu.SemaphoreType.DMA,
        pltpu.SemaphoreType.DMA,
        pltpu.SemaphoreType.DMA,
    ]


@functools.partial(
    pl.kernel,
    out_type=jax.ShapeDtypeStruct((N_EDGES, D_FEAT), jnp.float32),
    mesh=_mesh,
    scratch_types=_scratch,
)
def _edge_grad(x_hbm, negx_hbm, src_hbm, dst_hbm, out_hbm, *scr):
    idx_s = [scr[7 * b + 0] for b in range(NBUF)]
    idx_d = [scr[7 * b + 1] for b in range(NBUF)]
    rows = [scr[7 * b + 2] for b in range(NBUF)]
    sem_i = [scr[7 * b + 3] for b in range(NBUF)]
    sem_g = [scr[7 * b + 4] for b in range(NBUF)]
    sem_a = [scr[7 * b + 5] for b in range(NBUF)]
    sem_o = [scr[7 * b + 6] for b in range(NBUF)]

    wid = lax.axis_index("s") * 2 + lax.axis_index("c")
    base0 = wid * E_PER_W

    def group_body(g, carry):
        bases = [base0 + (g * NBUF + b) * C for b in range(NBUF)]
        d_i, d_g, d_a, d_o = [], [], [], []
        for b in range(NBUF):
            d_i.append((
                pltpu.async_copy(src_hbm.at[pl.ds(bases[b], C)], idx_s[b], sem_i[b]),
                pltpu.async_copy(dst_hbm.at[pl.ds(bases[b], C)], idx_d[b], sem_i[b]),
            ))
        for b in range(NBUF):
            d_i[b][0].wait()
            d_i[b][1].wait()
            d_g.append(pltpu.async_copy(x_hbm.at[idx_d[b]], rows[b], sem_g[b]))
        for b in range(NBUF):
            d_g[b].wait()
            d_a.append(
                pltpu.async_copy(negx_hbm.at[idx_s[b]], rows[b], sem_a[b], add=True))
        for b in range(NBUF):
            d_a[b].wait()
            d_o.append(
                pltpu.async_copy(rows[b], out_hbm.at[pl.ds(bases[b], C)], sem_o[b]))
        for b in range(NBUF):
            d_o[b].wait()
        return carry

    lax.fori_loop(0, NGROUP, group_body, 0, unroll=False)


def kernel(x, edge_index):
    negx = _negate(x)
    src = edge_index[0]
    dst = edge_index[1]
    return _edge_grad(x, negx, src, dst)


# speedup vs baseline: 7.3950x; 1.0255x over previous
import functools

import jax
import jax.numpy as jnp
from jax import lax
from jax.experimental import pallas as pl
from jax.experimental.pallas import tpu as pltpu
from jax.experimental.pallas import tpu_sc as plsc

N_NODES = 10000
N_EDGES = 320000
D_FEAT = 128

NW = 32
E_PER_W = N_EDGES // NW
C = 80
NCHUNK_W = E_PER_W // C
NBUF = 5
NGROUP = NCHUNK_W // NBUF

_mesh = plsc.VectorSubcoreMesh(core_axis_name="c", subcore_axis_name="s")


def _neg_body(x_ref, o_ref):
    o_ref[...] = -x_ref[...]


def _negate(x):
    return pl.pallas_call(
        _neg_body,
        out_shape=jax.ShapeDtypeStruct((N_NODES, D_FEAT), jnp.float32),
    )(x)


_scratch = [
    pltpu.VMEM((NCHUNK_W, C), jnp.int32),
    pltpu.VMEM((NCHUNK_W, C), jnp.int32),
]
for _b in range(NBUF):
    _scratch += [
        pltpu.VMEM((C, D_FEAT), jnp.float32),
        pltpu.SemaphoreType.DMA,
        pltpu.SemaphoreType.DMA,
        pltpu.SemaphoreType.DMA,
    ]


@functools.partial(
    pl.kernel,
    out_type=jax.ShapeDtypeStruct((N_EDGES, D_FEAT), jnp.float32),
    mesh=_mesh,
    scratch_types=_scratch,
)
def _edge_grad(x_hbm, negx_hbm, src_hbm, dst_hbm, out_hbm, idx_s, idx_d, *scr):
    rows = [scr[4 * b + 0] for b in range(NBUF)]
    sem_g = [scr[4 * b + 1] for b in range(NBUF)]
    sem_a = [scr[4 * b + 2] for b in range(NBUF)]
    sem_o = [scr[4 * b + 3] for b in range(NBUF)]

    wid = lax.axis_index("s") * 2 + lax.axis_index("c")
    base0 = wid * E_PER_W
    pltpu.sync_copy(src_hbm.at[wid], idx_s)
    pltpu.sync_copy(dst_hbm.at[wid], idx_d)

    def group_body(g, carry):
        ks = [g * NBUF + b for b in range(NBUF)]
        d_g, d_a, d_o = [], [], []
        for b in range(NBUF):
            d_g.append(
                pltpu.async_copy(x_hbm.at[idx_d.at[ks[b]]], rows[b], sem_g[b]))
        for b in range(NBUF):
            d_g[b].wait()
            d_a.append(
                pltpu.async_copy(negx_hbm.at[idx_s.at[ks[b]]], rows[b], sem_a[b],
                                 add=True))
        for b in range(NBUF):
            d_a[b].wait()
            d_o.append(
                pltpu.async_copy(rows[b], out_hbm.at[pl.ds(base0 + ks[b] * C, C)],
                                 sem_o[b]))
        for b in range(NBUF):
            d_o[b].wait()
        return carry

    lax.fori_loop(0, NGROUP, group_body, 0, unroll=False)


def kernel(x, edge_index):
    negx = _negate(x)
    src = edge_index[0].reshape(NW, NCHUNK_W, C)
    dst = edge_index[1].reshape(NW, NCHUNK_W, C)
    return _edge_grad(x, negx, src, dst)
